# Initial kernel scaffold; baseline (speedup 1.0000x reference)
import jax
import jax.numpy as jnp
from jax.experimental import pallas as pl


def kernel(x_scalar, k_dot_r, sinc_damping, batch, down_projection, W_pre1, W_pre2, gamma, beta, W_up, W_upd1, W_upd2):
    raise NotImplementedError("write your pallas kernel here")


# speedup vs baseline: 12.4088x; 12.4088x over previous
import functools

import jax
import jax.numpy as jnp
from jax.experimental import pallas as pl
from jax.experimental.pallas import tpu as pltpu

N = 10000
K = 32
D = 128
P = 8
B = 256

A = 400
NCHUNK = N // A
KD = K * D


def _silu(x):
    return x * jax.nn.sigmoid(x)


def _phase_a_kernel(x_ref, kdr_ref, sinc_ref, batch_ref,
                    w1t_ref, w2t_ref, gamma_ref, beta_ref, kfilt_ref,
                    gr_ref, gi_ref, c_ref, s_ref,
                    sfr_acc, sfi_acc):
    i = pl.program_id(0)

    @pl.when(i == 0)
    def _init():
        sfr_acc[...] = jnp.zeros_like(sfr_acc)
        sfi_acc[...] = jnp.zeros_like(sfi_acc)

    x = x_ref[...]
    h = _silu(jnp.dot(x, w1t_ref[...], preferred_element_type=jnp.float32))
    h = _silu(jnp.dot(h, w2t_ref[...], preferred_element_type=jnp.float32))
    xr = x + h
    mean = jnp.mean(xr, axis=-1, keepdims=True)
    var = jnp.mean((xr - mean) ** 2, axis=-1, keepdims=True)
    xr = (xr - mean) * jax.lax.rsqrt(var + 1e-5) * gamma_ref[...] + beta_ref[...]

    sinc = sinc_ref[...]
    kdr = kdr_ref[...]
    c = jnp.cos(kdr) * sinc
    s = jnp.sin(kdr) * sinc
    c_ref[...] = c.astype(jnp.bfloat16)
    s_ref[...] = s.astype(jnp.bfloat16)

    xrb = xr.astype(jnp.bfloat16)
    cb = c.astype(jnp.bfloat16)
    sb = s.astype(jnp.bfloat16)
    zr = jnp.concatenate([cb[:, k:k + 1] * xrb for k in range(K)], axis=1)
    zi = jnp.concatenate([sb[:, k:k + 1] * xrb for k in range(K)], axis=1)

    bvec = batch_ref[0]
    ot = (jax.lax.broadcasted_iota(jnp.int32, (B, A), 0) == bvec
          ).astype(jnp.bfloat16)
    sfr_acc[...] += jax.lax.dot(ot, zr, preferred_element_type=jnp.float32)
    sfi_acc[...] += jax.lax.dot(ot, zi, preferred_element_type=jnp.float32)

    @pl.when(i == NCHUNK - 1)
    def _emit():
        kf = kfilt_ref[...]
        gr_ref[...] = (kf * sfr_acc[...]).astype(jnp.bfloat16)
        gi_ref[...] = (kf * sfi_acc[...]).astype(jnp.bfloat16)


def _phase_b_kernel(x_ref, batch_ref, c_ref, s_ref, gr_ref, gi_ref,
                    wu1t_ref, wu2t_ref, out_ref):
    bvec = batch_ref[0]
    ot = (jax.lax.broadcasted_iota(jnp.int32, (B, A), 0) == bvec
          ).astype(jnp.bfloat16)
    garr = jax.lax.dot_general(ot, gr_ref[...], (((0,), (0,)), ((), ())),
                               preferred_element_type=jnp.float32)
    gari = jax.lax.dot_general(ot, gi_ref[...], (((0,), (0,)), ((), ())),
                               preferred_element_type=jnp.float32)
    c = c_ref[...].astype(jnp.float32)
    s = s_ref[...].astype(jnp.float32)
    ew = jnp.zeros((A, D), dtype=jnp.float32)
    for k in range(K):
        ew += c[:, k:k + 1] * garr[:, k * D:(k + 1) * D]
        ew += s[:, k:k + 1] * gari[:, k * D:(k + 1) * D]
    x_new = x_ref[...] + ew
    u = _silu(jnp.dot(x_new, wu1t_ref[...], preferred_element_type=jnp.float32))
    u = _silu(jnp.dot(u, wu2t_ref[...], preferred_element_type=jnp.float32))
    out_ref[...] = x_new + u


@jax.jit
def kernel(x_scalar, k_dot_r, sinc_damping, batch, down_projection,
           W_pre1, W_pre2, gamma, beta, W_up, W_upd1, W_upd2):
    batch3 = batch.reshape(NCHUNK, 1, A)
    kfilt = (down_projection @ W_up.T).reshape(1, KD)
    gamma2 = gamma.reshape(1, D)
    beta2 = beta.reshape(1, D)

    chunk = lambda i: (i, 0)
    whole = lambda i: (0, 0)

    gr, gi, c_all, s_all = pl.pallas_call(
        _phase_a_kernel,
        grid=(NCHUNK,),
        in_specs=[
            pl.BlockSpec((A, D), chunk),
            pl.BlockSpec((A, K), chunk),
            pl.BlockSpec((A, K), chunk),
            pl.BlockSpec((1, 1, A), lambda i: (i, 0, 0)),
            pl.BlockSpec((D, D), whole),
            pl.BlockSpec((D, D), whole),
            pl.BlockSpec((1, D), whole),
            pl.BlockSpec((1, D), whole),
            pl.BlockSpec((1, KD), whole),
        ],
        out_specs=[
            pl.BlockSpec((B, KD), whole),
            pl.BlockSpec((B, KD), whole),
            pl.BlockSpec((A, K), chunk),
            pl.BlockSpec((A, K), chunk),
        ],
        out_shape=[
            jax.ShapeDtypeStruct((B, KD), jnp.bfloat16),
            jax.ShapeDtypeStruct((B, KD), jnp.bfloat16),
            jax.ShapeDtypeStruct((N, K), jnp.bfloat16),
            jax.ShapeDtypeStruct((N, K), jnp.bfloat16),
        ],
        scratch_shapes=[
            pltpu.VMEM((B, KD), jnp.float32),
            pltpu.VMEM((B, KD), jnp.float32),
        ],
    )(x_scalar, k_dot_r, sinc_damping, batch3,
      W_pre1.T, W_pre2.T, gamma2, beta2, kfilt)

    out = pl.pallas_call(
        _phase_b_kernel,
        grid=(NCHUNK,),
        in_specs=[
            pl.BlockSpec((A, D), chunk),
            pl.BlockSpec((1, 1, A), lambda i: (i, 0, 0)),
            pl.BlockSpec((A, K), chunk),
            pl.BlockSpec((A, K), chunk),
            pl.BlockSpec((B, KD), whole),
            pl.BlockSpec((B, KD), whole),
            pl.BlockSpec((D, D), whole),
            pl.BlockSpec((D, D), whole),
        ],
        out_specs=pl.BlockSpec((A, D), chunk),
        out_shape=jax.ShapeDtypeStruct((N, D), jnp.float32),
    )(x_scalar, batch3, c_all, s_all, gr, gi, W_upd1.T, W_upd2.T)

    return out
